# baseline (speedup 1.0000x reference)
import functools
import math

import jax
import jax.numpy as jnp
from jax import lax
from jax.experimental import pallas as pl
from jax.experimental.pallas import tpu as pltpu
from jax.experimental.pallas import tpu_sc as plsc

D_MODEL = 64
_SCALE = math.sqrt(D_MODEL)


@functools.lru_cache(maxsize=None)
def _make_gather(vocab: int, d: int, b: int):
    info = plsc.get_sparse_core_info()
    nc, ns, lanes = info.num_cores, info.num_subcores, info.num_lanes
    nw = nc * ns
    assert b % nw == 0
    b_per_w = b // nw
    ch = 800
    while b_per_w % ch:
        ch //= 2
    n_ch = b_per_w // ch

    mesh = plsc.VectorSubcoreMesh(core_axis_name="c", subcore_axis_name="s")

    @functools.partial(
        pl.kernel,
        mesh=mesh,
        out_type=jax.ShapeDtypeStruct((b, d), jnp.float32),
        compiler_params=pltpu.CompilerParams(use_tc_tiling_on_sc=False),
        scratch_types=[
            pltpu.VMEM((b_per_w,), jnp.int32),
            pltpu.VMEM((ch, d), jnp.float32),
            pltpu.SemaphoreType.DMA,
        ],
    )
    def gather_kernel(idx_hbm, table_hbm, out_hbm, idx_v, rows_v, sem):
        wid = lax.axis_index("s") * nc + lax.axis_index("c")
        base = wid * b_per_w
        pltpu.sync_copy(idx_hbm.at[pl.ds(base, b_per_w)], idx_v)

        def chunk_body(c, carry):
            off = pl.multiple_of(c * ch, 8)
            pltpu.async_copy(
                table_hbm.at[idx_v.at[pl.ds(off, ch)]], rows_v, sem
            ).wait()

            def scale_body(r, carry2):
                for j in range(d // lanes):
                    sl = pl.ds(j * lanes, lanes)
                    rows_v[r, sl] = rows_v[r, sl] * _SCALE
                return carry2

            lax.fori_loop(0, ch, scale_body, 0)
            pltpu.sync_copy(rows_v, out_hbm.at[pl.ds(base + off, ch)])
            return carry

        lax.fori_loop(0, n_ch, chunk_body, 0)

    return gather_kernel


def kernel(x, table):
    batch, seq = x.shape
    vocab, d = table.shape
    b = batch * seq
    xf = x.reshape(b).astype(jnp.int32)
    out = _make_gather(vocab, d, b)(xf, table)
    return out.reshape(batch, seq, d)
